# Initial kernel scaffold; baseline (speedup 1.0000x reference)
import jax
import jax.numpy as jnp
from jax.experimental import pallas as pl


def kernel(x, edge_index, edge_weight, num_nodes, W_self, b_self, W_neigh, b_neigh):
    raise NotImplementedError("write your pallas kernel here")


# speedup vs baseline: 4.2599x; 4.2599x over previous
import functools

import jax
import jax.numpy as jnp
from jax import lax
from jax.experimental import pallas as pl
from jax.experimental.pallas import tpu as pltpu
from jax.experimental.pallas import tpu_sc as plsc

CH = 128
LANES = 16


@functools.lru_cache(maxsize=None)
def _make_sc_aggregate(n_nodes: int, d: int, n_edges: int):
    assert n_edges % CH == 0
    n_chunks = n_edges // CH
    nw = 32
    per = n_chunks // nw
    extra = n_chunks % nw
    rows_per_tile = (n_nodes // (16 * 8)) * 8
    hop = 104
    n_hops = rows_per_tile // hop
    assert n_hops * hop == rows_per_tile
    tail = n_nodes - 16 * rows_per_tile
    assert 0 <= tail <= CH and tail % 8 == 0
    vregs_per_row = d // LANES

    mesh = plsc.VectorSubcoreMesh(core_axis_name="c", subcore_axis_name="s")

    @functools.partial(
        pl.kernel,
        mesh=mesh,
        out_type=jax.ShapeDtypeStruct((2, n_nodes, d), jnp.float32),
        scratch_types=[
            pltpu.VMEM((CH, d), jnp.float32),
            pltpu.VMEM((CH,), jnp.int32),
            pltpu.VMEM((CH,), jnp.int32),
            pltpu.VMEM((CH,), jnp.float32),
            pltpu.VMEM_SHARED((n_nodes, d), jnp.float32),
            pltpu.SemaphoreType.DMA,
        ],
    )
    def sc_agg(row_hbm, col_hbm, w_hbm, x_hbm, out_hbm,
               rows_v, col_v, dst_v, w_v, accum, sem):
        c = lax.axis_index("c")
        s = lax.axis_index("s")
        wid = s * 2 + c

        zero16 = jnp.zeros((LANES,), jnp.float32)

        def _zero_row(i, _):
            for j in range(vregs_per_row):
                rows_v[i, pl.ds(j * LANES, LANES)] = zero16
            return 0

        lax.fori_loop(0, CH, _zero_row, 0)
        for h in range(n_hops):
            pltpu.sync_copy(rows_v.at[pl.ds(0, hop)],
                            accum.at[pl.ds(s * rows_per_tile + h * hop, hop)])
        if tail:
            @pl.when(s == 15)
            def _():
                pltpu.sync_copy(rows_v.at[pl.ds(0, tail)],
                                accum.at[pl.ds(16 * rows_per_tile, tail)])
        plsc.subcore_barrier()

        def _process(cidx):
            base = cidx * CH
            pltpu.sync_copy(col_hbm.at[pl.ds(base, CH)], col_v)
            pltpu.sync_copy(row_hbm.at[pl.ds(base, CH)], dst_v)
            pltpu.sync_copy(w_hbm.at[pl.ds(base, CH)], w_v)
            pltpu.async_copy(x_hbm.at[col_v], rows_v, sem).wait()

            def _scale(g, _):
                w16 = w_v[pl.ds(g * LANES, LANES)]
                for lane in range(LANES):
                    e = g * LANES + lane
                    wvec = jnp.full((LANES,), w16[lane], jnp.float32)
                    for j in range(vregs_per_row):
                        rows_v[e, pl.ds(j * LANES, LANES)] = (
                            rows_v[e, pl.ds(j * LANES, LANES)] * wvec)
                return 0

            lax.fori_loop(0, CH // LANES, _scale, 0)
            pltpu.sync_copy(rows_v, accum.at[dst_v], add=True)

        def _body(k, _):
            _process(wid + k * nw)
            return 0

        lax.fori_loop(0, per, _body, 0)

        @pl.when(wid < extra)
        def _():
            _process(per * nw + wid)

        plsc.subcore_barrier()

        for h in range(n_hops):
            r0 = s * rows_per_tile + h * hop
            pltpu.sync_copy(accum.at[pl.ds(r0, hop)], rows_v.at[pl.ds(0, hop)])
            pltpu.sync_copy(rows_v.at[pl.ds(0, hop)], out_hbm.at[c, pl.ds(r0, hop)])
        if tail:
            @pl.when(s == 15)
            def _():
                r0 = 16 * rows_per_tile
                pltpu.sync_copy(accum.at[pl.ds(r0, tail)], rows_v.at[pl.ds(0, tail)])
                pltpu.sync_copy(rows_v.at[pl.ds(0, tail)], out_hbm.at[c, pl.ds(r0, tail)])

    return sc_agg


def _tc_body(x_ref, a_ref, ws_ref, wn_ref, b_ref, o_ref):
    xb = x_ref[...]
    ab = a_ref[0] + a_ref[1]
    dn = (((1,), (1,)), ((), ()))
    o_ref[...] = (
        lax.dot_general(xb, ws_ref[...], dn, preferred_element_type=jnp.float32)
        + lax.dot_general(ab, wn_ref[...], dn, preferred_element_type=jnp.float32)
        + b_ref[...]
    )


@functools.lru_cache(maxsize=None)
def _make_tc_linear(n_nodes: int, d: int):
    br = 1000
    assert n_nodes % br == 0
    grid = (n_nodes // br,)
    return pl.pallas_call(
        _tc_body,
        grid=grid,
        in_specs=[
            pl.BlockSpec((br, d), lambda i: (i, 0)),
            pl.BlockSpec((2, br, d), lambda i: (0, i, 0)),
            pl.BlockSpec((d, d), lambda i: (0, 0)),
            pl.BlockSpec((d, d), lambda i: (0, 0)),
            pl.BlockSpec((1, d), lambda i: (0, 0)),
        ],
        out_specs=pl.BlockSpec((br, d), lambda i: (i, 0)),
        out_shape=jax.ShapeDtypeStruct((n_nodes, d), jnp.float32),
    )


def kernel(x, edge_index, edge_weight, num_nodes, W_self, b_self, W_neigh, b_neigh):
    n, d = x.shape
    e = edge_index.shape[1]
    ei = edge_index.astype(jnp.int32)
    row = (ei[0] % jnp.asarray(num_nodes, jnp.int32)).astype(jnp.int32)
    col = ei[1]
    agg = _make_sc_aggregate(n, d, e)(row, col, edge_weight.astype(jnp.float32), x)
    bias = (b_self + b_neigh).reshape(1, d).astype(jnp.float32)
    return _make_tc_linear(n, d)(x, agg, W_self, W_neigh, bias)
